# baseline (speedup 1.0000x reference)
import functools

import jax
import jax.numpy as jnp
from jax import lax
from jax.experimental import pallas as pl
from jax.experimental.pallas import tpu as pltpu
from jax.experimental.pallas import tpu_sc as plsc

NC = 2
NS = 16
EB = 128
GRP = 8
F = 128


def _sizes(N, E):
    tiles = NC * NS
    bpt = -(-E // (tiles * EB))
    bpt = -(-bpt // (2 * GRP)) * (2 * GRP)
    E_pad = tiles * bpt * EB
    N_pad = -(-(N + 1) // (8 * tiles)) * (8 * tiles)
    return N_pad, E_pad, bpt


def _sc_mesh():
    return plsc.VectorSubcoreMesh(
        core_axis_name="c", subcore_axis_name="s", num_cores=NC, num_subcores=NS
    )


def _make_deg(N_pad, nblk):
    bpt = nblk // (NC * NS)

    @functools.partial(
        pl.kernel,
        out_type=jax.ShapeDtypeStruct((NC * NS, N_pad), jnp.float32),
        mesh=_sc_mesh(),
        compiler_params=pltpu.CompilerParams(needs_layout_passes=False),
        scratch_types=[
            pltpu.VMEM((GRP, EB), jnp.int32),
            pltpu.VMEM((N_pad,), jnp.float32),
        ],
    )
    def deg_kernel(dstb, out, idx_d, acc):
        c = lax.axis_index("c")
        s = lax.axis_index("s")
        wid = c * NS + s
        ones = jnp.ones((16,), jnp.float32)
        zf = jnp.zeros((16,), jnp.float32)

        def zero(i, _):
            acc[pl.ds(i * 16, 16)] = zf
            return _

        lax.fori_loop(0, N_pad // 16, zero, None)

        def group(g, _):
            pltpu.sync_copy(dstb.at[pl.ds((g * NC * NS + wid) * GRP, GRP)], idx_d)
            for j in range(GRP):
                for q in range(EB // 16):
                    iv = idx_d[j, pl.ds(q * 16, 16)]
                    plsc.addupdate_scatter(acc, [iv], ones)
            return _

        lax.fori_loop(0, bpt // GRP, group, None)
        pltpu.sync_copy(acc, out.at[wid])

    return deg_kernel


def _make_spmm(N_pad, nblk):
    bpt = nblk // (NC * NS)
    rpt = N_pad // NS
    npair = bpt // (2 * GRP)

    @functools.partial(
        pl.kernel,
        out_type=jax.ShapeDtypeStruct((NC, N_pad, F), jnp.float32),
        mesh=_sc_mesh(),
        scratch_types=[
            pltpu.VMEM((2, GRP, EB), jnp.int32),
            pltpu.VMEM((2, GRP, EB), jnp.int32),
            pltpu.VMEM((2, EB, F), jnp.float32),
            pltpu.VMEM_SHARED((N_pad, F), jnp.float32),
            pltpu.SemaphoreType.DMA,
            pltpu.SemaphoreType.DMA,
            pltpu.SemaphoreType.DMA,
            pltpu.SemaphoreType.DMA,
        ],
    )
    def spmm_kernel(xs, srcb, dstb, out, idx_s, idx_d, rows, acc,
                    g0, g1, s0, s1):
        c = lax.axis_index("c")
        s = lax.axis_index("s")
        wid = c * NS + s
        base = wid * bpt
        gsem = (g0, g1)
        ssem = (s0, s1)

        def zero_row(i, _):
            for j in range(F // 16):
                rows[0, i, pl.ds(j * 16, 16)] = jnp.zeros((16,), jnp.float32)
            return _

        lax.fori_loop(0, EB, zero_row, None)
        for k in range(rpt // EB):
            pltpu.sync_copy(rows.at[0], acc.at[pl.ds(s * rpt + k * EB, EB)])
        plsc.subcore_barrier()

        def load_idx(p, g):
            pltpu.sync_copy(srcb.at[pl.ds(base + g * GRP, GRP)], idx_s.at[p])
            pltpu.sync_copy(dstb.at[pl.ds(base + g * GRP, GRP)], idx_d.at[p])

        def start_g(p, jj, b):
            pltpu.async_copy(xs.at[idx_s.at[p, jj]], rows.at[b], gsem[b])

        def wait_g(p, jj, b):
            pltpu.make_async_copy(
                xs.at[idx_s.at[p, jj]], rows.at[b], gsem[b]).wait()

        def start_s(p, jj, b):
            pltpu.async_copy(rows.at[b], acc.at[idx_d.at[p, jj]], ssem[b],
                             add=True)

        def wait_s(p, jj, b):
            pltpu.make_async_copy(
                rows.at[b], acc.at[idx_d.at[p, jj]], ssem[b]).wait()

        load_idx(0, 0)

        def pair(dp, _):
            for sjj in range(2 * GRP):
                p, jj = divmod(sjj, GRP)
                b = sjj & 1
                if sjj >= 2:
                    wait_s(divmod(sjj - 2, GRP)[0], (sjj - 2) % GRP, b)
                else:
                    @pl.when(dp > 0)
                    def _():
                        wait_s(1, GRP - 2 + sjj, b)
                start_g(p, jj, b)
                if sjj >= 1:
                    pp, pjj = divmod(sjj - 1, GRP)
                    wait_g(pp, pjj, 1 - b)
                    start_s(pp, pjj, 1 - b)
                else:
                    @pl.when(dp > 0)
                    def _():
                        wait_g(1, GRP - 1, 1 - b)
                        start_s(1, GRP - 1, 1 - b)
                if sjj == 2:
                    load_idx(1, 2 * dp + 1)
                if sjj == GRP + 2:
                    @pl.when(dp < npair - 1)
                    def _():
                        load_idx(0, 2 * dp + 2)
            return _

        lax.fori_loop(0, npair, pair, None)
        wait_g(1, GRP - 1, 1)
        start_s(1, GRP - 1, 1)
        wait_s(1, GRP - 2, 0)
        wait_s(1, GRP - 1, 1)
        plsc.subcore_barrier()
        pltpu.sync_copy(acc.at[pl.ds(s * rpt, rpt)], out.at[c].at[pl.ds(s * rpt, rpt)])

    return spmm_kernel


def _prescale_body(deg_ref, x_ref, xs_ref, dinv_ref):
    d = jnp.sum(jnp.transpose(deg_ref[...]), axis=1, keepdims=True)
    dinv = jnp.where(d > 0.0, lax.rsqrt(d), 0.0)
    dinv_ref[...] = dinv
    xs_ref[...] = x_ref[...] * dinv


def _combine1_body(dinv_ref, g_ref, x1_ref, xs1_ref):
    dinv = dinv_ref[...]
    x1 = -(g_ref[0] + g_ref[1]) * dinv
    x1_ref[...] = x1
    xs1_ref[...] = x1 * dinv


def _layer_pre_body(x0_ref, x1_ref, w_ref, b_ref, pre_ref):
    acc = jnp.dot(x0_ref[...], w_ref[0:F, :], preferred_element_type=jnp.float32)
    acc += jnp.dot(x1_ref[...], w_ref[F:2 * F, :], preferred_element_type=jnp.float32)
    pre_ref[...] = acc + b_ref[...]


def _layer_post_body(dinv_ref, g2_ref, x0_ref, pre_ref, w_ref, h_ref, xsh_ref, *, relu):
    dinv = dinv_ref[...]
    x2 = -2.0 * (g2_ref[0] + g2_ref[1]) * dinv - x0_ref[...]
    acc = pre_ref[...] + jnp.dot(x2, w_ref[2 * F:3 * F, :],
                                 preferred_element_type=jnp.float32)
    if relu:
        acc = jnp.maximum(acc, 0.0)
    h_ref[...] = acc
    if xsh_ref is not None:
        xsh_ref[...] = acc * dinv


def _tc_grid(N_pad):
    nb = 8
    B = N_pad // nb
    dinv_spec = pl.BlockSpec((B, 1), lambda i: (i, 0))
    row_spec = pl.BlockSpec((B, F), lambda i: (i, 0))
    g_spec = pl.BlockSpec((NC, B, F), lambda i: (0, i, 0))
    return nb, B, dinv_spec, row_spec, g_spec


def _prescale(deg32, x_pad):
    N_pad = x_pad.shape[0]
    nb, B, dinv_spec, row_spec, g_spec = _tc_grid(N_pad)
    deg_spec = pl.BlockSpec((NC * NS, B), lambda i: (0, i))
    return pl.pallas_call(
        _prescale_body,
        grid=(nb,),
        in_specs=[deg_spec, row_spec],
        out_specs=(row_spec, dinv_spec),
        out_shape=(
            jax.ShapeDtypeStruct((N_pad, F), jnp.float32),
            jax.ShapeDtypeStruct((N_pad, 1), jnp.float32),
        ),
    )(deg32, x_pad)


def _combine1(dinv, g):
    N_pad = g.shape[1]
    nb, B, dinv_spec, row_spec, g_spec = _tc_grid(N_pad)
    return pl.pallas_call(
        _combine1_body,
        grid=(nb,),
        in_specs=[dinv_spec, g_spec],
        out_specs=(row_spec, row_spec),
        out_shape=(
            jax.ShapeDtypeStruct((N_pad, F), jnp.float32),
            jax.ShapeDtypeStruct((N_pad, F), jnp.float32),
        ),
    )(dinv, g)


def _layer_pre(x0, x1, w, b):
    N_pad = x0.shape[0]
    nb, B, dinv_spec, row_spec, g_spec = _tc_grid(N_pad)
    w_spec = pl.BlockSpec((3 * F, F), lambda i: (0, 0))
    b_spec = pl.BlockSpec((1, F), lambda i: (0, 0))
    return pl.pallas_call(
        _layer_pre_body,
        grid=(nb,),
        in_specs=[row_spec, row_spec, w_spec, b_spec],
        out_specs=row_spec,
        out_shape=jax.ShapeDtypeStruct((N_pad, F), jnp.float32),
    )(x0, x1, w, b)


def _layer_post(dinv, g2, x0, pre, w, *, relu, need_xs):
    N_pad = x0.shape[0]
    nb, B, dinv_spec, row_spec, g_spec = _tc_grid(N_pad)
    w_spec = pl.BlockSpec((3 * F, F), lambda i: (0, 0))
    if need_xs:
        body = functools.partial(_layer_post_body, relu=relu)
        out_specs = (row_spec, row_spec)
        out_shape = (
            jax.ShapeDtypeStruct((N_pad, F), jnp.float32),
            jax.ShapeDtypeStruct((N_pad, F), jnp.float32),
        )
    else:
        def body(dinv_ref, g2_ref, x0_ref, pre_ref, w_ref, h_ref):
            _layer_post_body(dinv_ref, g2_ref, x0_ref, pre_ref, w_ref, h_ref,
                             None, relu=relu)
        out_specs = row_spec
        out_shape = jax.ShapeDtypeStruct((N_pad, F), jnp.float32)
    return pl.pallas_call(
        body,
        grid=(nb,),
        in_specs=[dinv_spec, g_spec, row_spec, row_spec, w_spec],
        out_specs=out_specs,
        out_shape=out_shape,
    )(dinv, g2, x0, pre, w)


def kernel(features, edge_index, W1, b1, W2, b2):
    N, D = features.shape
    E = edge_index.shape[1]
    N_pad, E_pad, bpt = _sizes(N, E)
    nblk = E_pad // EB

    src = edge_index[0]
    dst = edge_index[1]
    pad = N + (jnp.arange(E_pad - E, dtype=jnp.int32) % (N_pad - N))
    srcb = jnp.concatenate([src, pad]).reshape(nblk, EB)
    dstb = jnp.concatenate([dst, pad]).reshape(nblk, EB)
    x_pad = jnp.pad(features, ((0, N_pad - N), (0, 0)))

    deg_fn = _make_deg(N_pad, nblk)
    spmm_fn = _make_spmm(N_pad, nblk)

    deg32 = deg_fn(dstb)
    xs, dinv = _prescale(deg32, x_pad)
    g = spmm_fn(xs, srcb, dstb)
    x1, xs1 = _combine1(dinv, g)
    pre1 = _layer_pre(x_pad, x1, W1, b1.reshape(1, F))
    g2 = spmm_fn(xs1, srcb, dstb)
    h, xsh = _layer_post(dinv, g2, x_pad, pre1, W1, relu=True, need_xs=True)
    gh = spmm_fn(xsh, srcb, dstb)
    h1, xsh1 = _combine1(dinv, gh)
    pre2 = _layer_pre(h, h1, W2, b2.reshape(1, F))
    gh2 = spmm_fn(xsh1, srcb, dstb)
    out = _layer_post(dinv, gh2, h, pre2, W2, relu=False, need_xs=False)
    return out[:N]


# speedup vs baseline: 1.0020x; 1.0020x over previous
import functools

import jax
import jax.numpy as jnp
from jax import lax
from jax.experimental import pallas as pl
from jax.experimental.pallas import tpu as pltpu
from jax.experimental.pallas import tpu_sc as plsc

NC = 2
NS = 16
EB = 128
GRP = 8
F = 128


def _sizes(N, E):
    tiles = NC * NS
    bpt = -(-E // (tiles * EB))
    bpt = -(-bpt // (2 * GRP)) * (2 * GRP)
    E_pad = tiles * bpt * EB
    N_pad = -(-(N + 1) // (8 * tiles)) * (8 * tiles)
    return N_pad, E_pad, bpt


def _sc_mesh():
    return plsc.VectorSubcoreMesh(
        core_axis_name="c", subcore_axis_name="s", num_cores=NC, num_subcores=NS
    )


def _make_deg(N_pad, nblk):
    bpt = nblk // (NC * NS)

    @functools.partial(
        pl.kernel,
        out_type=jax.ShapeDtypeStruct((NC * NS, N_pad), jnp.float32),
        mesh=_sc_mesh(),
        compiler_params=pltpu.CompilerParams(needs_layout_passes=False),
        scratch_types=[
            pltpu.VMEM((GRP, EB), jnp.int32),
            pltpu.VMEM((N_pad,), jnp.float32),
        ],
    )
    def deg_kernel(dstb, out, idx_d, acc):
        c = lax.axis_index("c")
        s = lax.axis_index("s")
        wid = c * NS + s
        ones = jnp.ones((16,), jnp.float32)
        zf = jnp.zeros((16,), jnp.float32)

        def zero(i, _):
            acc[pl.ds(i * 16, 16)] = zf
            return _

        lax.fori_loop(0, N_pad // 16, zero, None)

        def group(g, _):
            pltpu.sync_copy(dstb.at[pl.ds((g * NC * NS + wid) * GRP, GRP)], idx_d)
            for j in range(GRP):
                for q in range(EB // 16):
                    iv = idx_d[j, pl.ds(q * 16, 16)]
                    plsc.addupdate_scatter(acc, [iv], ones)
            return _

        lax.fori_loop(0, bpt // GRP, group, None)
        pltpu.sync_copy(acc, out.at[wid])

    return deg_kernel


def _make_spmm(N_pad, nblk):
    bpt = nblk // (NC * NS)
    rpt = N_pad // NS
    npair = bpt // (2 * GRP)

    @functools.partial(
        pl.kernel,
        out_type=jax.ShapeDtypeStruct((NC, N_pad, F), jnp.float32),
        mesh=_sc_mesh(),
        scratch_types=[
            pltpu.VMEM((2, GRP, EB), jnp.int32),
            pltpu.VMEM((2, GRP, EB), jnp.int32),
            pltpu.VMEM((2, EB, F), jnp.float32),
            pltpu.VMEM_SHARED((N_pad, F), jnp.float32),
            pltpu.SemaphoreType.DMA,
            pltpu.SemaphoreType.DMA,
            pltpu.SemaphoreType.DMA,
            pltpu.SemaphoreType.DMA,
        ],
    )
    def spmm_kernel(xs, srcb, dstb, out, idx_s, idx_d, rows, acc,
                    g0, g1, s0, s1):
        c = lax.axis_index("c")
        s = lax.axis_index("s")
        wid = c * NS + s
        base = wid * bpt
        gsem = (g0, g1)
        ssem = (s0, s1)

        def zero_row(i, _):
            for j in range(F // 16):
                rows[0, i, pl.ds(j * 16, 16)] = jnp.zeros((16,), jnp.float32)
            return _

        lax.fori_loop(0, EB, zero_row, None)
        for k in range(rpt // EB):
            pltpu.sync_copy(rows.at[0], acc.at[pl.ds(s * rpt + k * EB, EB)])
        plsc.subcore_barrier()

        def load_idx(p, g):
            pltpu.sync_copy(srcb.at[pl.ds(base + g * GRP, GRP)], idx_s.at[p])
            pltpu.sync_copy(dstb.at[pl.ds(base + g * GRP, GRP)], idx_d.at[p])

        def start_g(p, jj, b):
            pltpu.async_copy(xs.at[idx_s.at[p, jj]], rows.at[b], gsem[b])

        def wait_g(p, jj, b):
            pltpu.make_async_copy(
                xs.at[idx_s.at[p, jj]], rows.at[b], gsem[b]).wait()

        def start_s(p, jj, b):
            pltpu.async_copy(rows.at[b], acc.at[idx_d.at[p, jj]], ssem[b],
                             add=True)

        def wait_s(p, jj, b):
            pltpu.make_async_copy(
                rows.at[b], acc.at[idx_d.at[p, jj]], ssem[b]).wait()

        load_idx(0, 0)

        def pair(dp, _):
            for sjj in range(2 * GRP):
                p, jj = divmod(sjj, GRP)
                b = sjj & 1
                if sjj >= 2:
                    wait_s(divmod(sjj - 2, GRP)[0], (sjj - 2) % GRP, b)
                else:
                    @pl.when(dp > 0)
                    def _():
                        wait_s(1, GRP - 2 + sjj, b)
                start_g(p, jj, b)
                if sjj >= 1:
                    pp, pjj = divmod(sjj - 1, GRP)
                    wait_g(pp, pjj, 1 - b)
                    start_s(pp, pjj, 1 - b)
                else:
                    @pl.when(dp > 0)
                    def _():
                        wait_g(1, GRP - 1, 1 - b)
                        start_s(1, GRP - 1, 1 - b)
                if sjj == 2:
                    load_idx(1, 2 * dp + 1)
                if sjj == GRP + 2:
                    @pl.when(dp < npair - 1)
                    def _():
                        load_idx(0, 2 * dp + 2)
            return _

        lax.fori_loop(0, npair, pair, None)
        wait_g(1, GRP - 1, 1)
        start_s(1, GRP - 1, 1)
        wait_s(1, GRP - 2, 0)
        wait_s(1, GRP - 1, 1)
        plsc.subcore_barrier()
        pltpu.sync_copy(acc.at[pl.ds(s * rpt, rpt)], out.at[c].at[pl.ds(s * rpt, rpt)])

    return spmm_kernel


def _prescale_body(deg_ref, x_ref, xs_ref, dinv_ref):
    d = jnp.sum(jnp.transpose(deg_ref[...]), axis=1, keepdims=True)
    dinv = jnp.where(d > 0.0, lax.rsqrt(d), 0.0)
    dinv_ref[...] = dinv
    xs_ref[...] = x_ref[...] * dinv


def _combine_pre_body(dinv_ref, g_ref, x0_ref, w_ref, b_ref, xs1_ref, pre_ref):
    dinv = dinv_ref[...]
    x1 = -(g_ref[0] + g_ref[1]) * dinv
    xs1_ref[...] = x1 * dinv
    acc = jnp.dot(x0_ref[...], w_ref[0:F, :], preferred_element_type=jnp.float32)
    acc += jnp.dot(x1, w_ref[F:2 * F, :], preferred_element_type=jnp.float32)
    pre_ref[...] = acc + b_ref[...]


def _layer_post_body(dinv_ref, g2_ref, x0_ref, pre_ref, w_ref, h_ref, xsh_ref, *, relu):
    dinv = dinv_ref[...]
    x2 = -2.0 * (g2_ref[0] + g2_ref[1]) * dinv - x0_ref[...]
    acc = pre_ref[...] + jnp.dot(x2, w_ref[2 * F:3 * F, :],
                                 preferred_element_type=jnp.float32)
    if relu:
        acc = jnp.maximum(acc, 0.0)
    h_ref[...] = acc
    if xsh_ref is not None:
        xsh_ref[...] = acc * dinv


def _tc_grid(N_pad):
    nb = 8
    B = N_pad // nb
    dinv_spec = pl.BlockSpec((B, 1), lambda i: (i, 0))
    row_spec = pl.BlockSpec((B, F), lambda i: (i, 0))
    g_spec = pl.BlockSpec((NC, B, F), lambda i: (0, i, 0))
    return nb, B, dinv_spec, row_spec, g_spec


def _prescale(deg32, x_pad):
    N_pad = x_pad.shape[0]
    nb, B, dinv_spec, row_spec, g_spec = _tc_grid(N_pad)
    deg_spec = pl.BlockSpec((NC * NS, B), lambda i: (0, i))
    return pl.pallas_call(
        _prescale_body,
        grid=(nb,),
        in_specs=[deg_spec, row_spec],
        out_specs=(row_spec, dinv_spec),
        out_shape=(
            jax.ShapeDtypeStruct((N_pad, F), jnp.float32),
            jax.ShapeDtypeStruct((N_pad, 1), jnp.float32),
        ),
    )(deg32, x_pad)


def _combine_pre(dinv, g, x0, w, b):
    N_pad = g.shape[1]
    nb, B, dinv_spec, row_spec, g_spec = _tc_grid(N_pad)
    w_spec = pl.BlockSpec((3 * F, F), lambda i: (0, 0))
    b_spec = pl.BlockSpec((1, F), lambda i: (0, 0))
    return pl.pallas_call(
        _combine_pre_body,
        grid=(nb,),
        in_specs=[dinv_spec, g_spec, row_spec, w_spec, b_spec],
        out_specs=(row_spec, row_spec),
        out_shape=(
            jax.ShapeDtypeStruct((N_pad, F), jnp.float32),
            jax.ShapeDtypeStruct((N_pad, F), jnp.float32),
        ),
    )(dinv, g, x0, w, b)


def _layer_post(dinv, g2, x0, pre, w, *, relu, need_xs):
    N_pad = x0.shape[0]
    nb, B, dinv_spec, row_spec, g_spec = _tc_grid(N_pad)
    w_spec = pl.BlockSpec((3 * F, F), lambda i: (0, 0))
    if need_xs:
        body = functools.partial(_layer_post_body, relu=relu)
        out_specs = (row_spec, row_spec)
        out_shape = (
            jax.ShapeDtypeStruct((N_pad, F), jnp.float32),
            jax.ShapeDtypeStruct((N_pad, F), jnp.float32),
        )
    else:
        def body(dinv_ref, g2_ref, x0_ref, pre_ref, w_ref, h_ref):
            _layer_post_body(dinv_ref, g2_ref, x0_ref, pre_ref, w_ref, h_ref,
                             None, relu=relu)
        out_specs = row_spec
        out_shape = jax.ShapeDtypeStruct((N_pad, F), jnp.float32)
    return pl.pallas_call(
        body,
        grid=(nb,),
        in_specs=[dinv_spec, g_spec, row_spec, row_spec, w_spec],
        out_specs=out_specs,
        out_shape=out_shape,
    )(dinv, g2, x0, pre, w)


def kernel(features, edge_index, W1, b1, W2, b2):
    N, D = features.shape
    E = edge_index.shape[1]
    N_pad, E_pad, bpt = _sizes(N, E)
    nblk = E_pad // EB

    src = edge_index[0]
    dst = edge_index[1]
    pad = N + (jnp.arange(E_pad - E, dtype=jnp.int32) % (N_pad - N))
    srcb = jnp.concatenate([src, pad]).reshape(nblk, EB)
    dstb = jnp.concatenate([dst, pad]).reshape(nblk, EB)
    x_pad = jnp.pad(features, ((0, N_pad - N), (0, 0)))

    deg_fn = _make_deg(N_pad, nblk)
    spmm_fn = _make_spmm(N_pad, nblk)

    deg32 = deg_fn(dstb)
    xs, dinv = _prescale(deg32, x_pad)
    g = spmm_fn(xs, srcb, dstb)
    xs1, pre1 = _combine_pre(dinv, g, x_pad, W1, b1.reshape(1, F))
    g2 = spmm_fn(xs1, srcb, dstb)
    h, xsh = _layer_post(dinv, g2, x_pad, pre1, W1, relu=True, need_xs=True)
    gh = spmm_fn(xsh, srcb, dstb)
    xsh1, pre2 = _combine_pre(dinv, gh, h, W2, b2.reshape(1, F))
    gh2 = spmm_fn(xsh1, srcb, dstb)
    out = _layer_post(dinv, gh2, h, pre2, W2, relu=False, need_xs=False)
    return out[:N]


# speedup vs baseline: 1.0465x; 1.0443x over previous
import functools

import jax
import jax.numpy as jnp
from jax import lax
from jax.experimental import pallas as pl
from jax.experimental.pallas import tpu as pltpu
from jax.experimental.pallas import tpu_sc as plsc

NC = 2
NS = 16
EB = 128
GRP = 8
F = 128


def _sizes(N, E):
    tiles = NC * NS
    bpt = -(-E // (tiles * EB))
    bpt = -(-bpt // (2 * GRP)) * (2 * GRP)
    E_pad = tiles * bpt * EB
    N_pad = -(-(N + 1) // (8 * tiles)) * (8 * tiles)
    return N_pad, E_pad, bpt


def _sc_mesh():
    return plsc.VectorSubcoreMesh(
        core_axis_name="c", subcore_axis_name="s", num_cores=NC, num_subcores=NS
    )


def _make_deg(N_pad, nblk):
    bpt = nblk // (NC * NS)

    @functools.partial(
        pl.kernel,
        out_type=jax.ShapeDtypeStruct((NC * NS, N_pad), jnp.float32),
        mesh=_sc_mesh(),
        compiler_params=pltpu.CompilerParams(needs_layout_passes=False),
        scratch_types=[
            pltpu.VMEM((2, GRP, EB), jnp.int32),
            pltpu.VMEM((N_pad,), jnp.float32),
            pltpu.SemaphoreType.DMA,
            pltpu.SemaphoreType.DMA,
        ],
    )
    def deg_kernel(dstb, out, idx_d, acc, i0, i1):
        c = lax.axis_index("c")
        s = lax.axis_index("s")
        wid = c * NS + s
        isem = (i0, i1)
        ones = jnp.ones((16,), jnp.float32)
        zf = jnp.zeros((16,), jnp.float32)
        npair = bpt // (2 * GRP)

        def load_idx(p, g):
            pltpu.async_copy(dstb.at[pl.ds((g * NC * NS + wid) * GRP, GRP)],
                             idx_d.at[p], isem[p])

        def wait_idx(p):
            pltpu.make_async_copy(dstb.at[pl.ds(0, GRP)], idx_d.at[p],
                                  isem[p]).wait()

        def scatter_group(p):
            for j in range(GRP):
                for q in range(EB // 16):
                    iv = idx_d[p, j, pl.ds(q * 16, 16)]
                    plsc.addupdate_scatter(acc, [iv], ones)

        load_idx(0, 0)

        def zero(i, _):
            acc[pl.ds(i * 16, 16)] = zf
            return _

        lax.fori_loop(0, N_pad // 16, zero, None)

        def pair(dp, _):
            wait_idx(0)
            load_idx(1, 2 * dp + 1)
            scatter_group(0)
            wait_idx(1)

            @pl.when(dp < npair - 1)
            def _():
                load_idx(0, 2 * dp + 2)
            scatter_group(1)
            return _

        lax.fori_loop(0, npair, pair, None)
        pltpu.sync_copy(acc, out.at[wid])

    return deg_kernel


def _make_spmm(N_pad, nblk):
    bpt = nblk // (NC * NS)
    rpt = N_pad // NS
    npair = bpt // (2 * GRP)

    @functools.partial(
        pl.kernel,
        out_type=jax.ShapeDtypeStruct((NC, N_pad, F), jnp.float32),
        mesh=_sc_mesh(),
        scratch_types=[
            pltpu.VMEM((2, GRP, EB), jnp.int32),
            pltpu.VMEM((2, GRP, EB), jnp.int32),
            pltpu.VMEM((2, EB, F), jnp.float32),
            pltpu.VMEM_SHARED((N_pad, F), jnp.float32),
            pltpu.SemaphoreType.DMA,
            pltpu.SemaphoreType.DMA,
            pltpu.SemaphoreType.DMA,
            pltpu.SemaphoreType.DMA,
            pltpu.SemaphoreType.DMA,
            pltpu.SemaphoreType.DMA,
        ],
    )
    def spmm_kernel(xs, srcb, dstb, out, idx_s, idx_d, rows, acc,
                    g0, g1, s0, s1, i0, i1):
        c = lax.axis_index("c")
        s = lax.axis_index("s")
        wid = c * NS + s
        base = wid * bpt
        gsem = (g0, g1)
        ssem = (s0, s1)
        isem = (i0, i1)

        def zero_row(i, _):
            for j in range(F // 16):
                rows[0, i, pl.ds(j * 16, 16)] = jnp.zeros((16,), jnp.float32)
            return _

        lax.fori_loop(0, EB, zero_row, None)
        for k in range(rpt // EB):
            pltpu.sync_copy(rows.at[0], acc.at[pl.ds(s * rpt + k * EB, EB)])
        plsc.subcore_barrier()

        def load_idx(p, g):
            pltpu.async_copy(srcb.at[pl.ds(base + g * GRP, GRP)], idx_s.at[p],
                             isem[p])
            pltpu.async_copy(dstb.at[pl.ds(base + g * GRP, GRP)], idx_d.at[p],
                             isem[p])

        def wait_idx(p):
            pltpu.make_async_copy(srcb.at[pl.ds(base, GRP)], idx_s.at[p],
                                  isem[p]).wait()
            pltpu.make_async_copy(dstb.at[pl.ds(base, GRP)], idx_d.at[p],
                                  isem[p]).wait()

        def start_g(p, jj, b):
            pltpu.async_copy(xs.at[idx_s.at[p, jj]], rows.at[b], gsem[b])

        def wait_g(p, jj, b):
            pltpu.make_async_copy(
                xs.at[idx_s.at[p, jj]], rows.at[b], gsem[b]).wait()

        def start_s(p, jj, b):
            pltpu.async_copy(rows.at[b], acc.at[idx_d.at[p, jj]], ssem[b],
                             add=True)

        def wait_s(p, jj, b):
            pltpu.make_async_copy(
                rows.at[b], acc.at[idx_d.at[p, jj]], ssem[b]).wait()

        load_idx(0, 0)

        def pair(dp, _):
            for sjj in range(2 * GRP):
                p, jj = divmod(sjj, GRP)
                b = sjj & 1
                if jj == 0:
                    wait_idx(p)
                if sjj >= 2:
                    wait_s(divmod(sjj - 2, GRP)[0], (sjj - 2) % GRP, b)
                else:
                    @pl.when(dp > 0)
                    def _():
                        wait_s(1, GRP - 2 + sjj, b)
                start_g(p, jj, b)
                if sjj >= 1:
                    pp, pjj = divmod(sjj - 1, GRP)
                    wait_g(pp, pjj, 1 - b)
                    start_s(pp, pjj, 1 - b)
                else:
                    @pl.when(dp > 0)
                    def _():
                        wait_g(1, GRP - 1, 1 - b)
                        start_s(1, GRP - 1, 1 - b)
                if sjj == 2:
                    load_idx(1, 2 * dp + 1)
                if sjj == GRP + 2:
                    @pl.when(dp < npair - 1)
                    def _():
                        load_idx(0, 2 * dp + 2)
            return _

        lax.fori_loop(0, npair, pair, None)
        wait_g(1, GRP - 1, 1)
        start_s(1, GRP - 1, 1)
        wait_s(1, GRP - 2, 0)
        wait_s(1, GRP - 1, 1)
        plsc.subcore_barrier()
        pltpu.sync_copy(acc.at[pl.ds(s * rpt, rpt)], out.at[c].at[pl.ds(s * rpt, rpt)])

    return spmm_kernel


def _prescale_body(deg_ref, x_ref, xs_ref, dinv_ref):
    d = jnp.sum(jnp.transpose(deg_ref[...]), axis=1, keepdims=True)
    dinv = jnp.where(d > 0.0, lax.rsqrt(d), 0.0)
    dinv_ref[...] = dinv
    xs_ref[...] = x_ref[...] * dinv


def _combine_pre_body(dinv_ref, g_ref, x0_ref, w_ref, b_ref, xs1_ref, pre_ref):
    dinv = dinv_ref[...]
    x1 = -(g_ref[0] + g_ref[1]) * dinv
    xs1_ref[...] = x1 * dinv
    acc = jnp.dot(x0_ref[...], w_ref[0:F, :], preferred_element_type=jnp.float32)
    acc += jnp.dot(x1, w_ref[F:2 * F, :], preferred_element_type=jnp.float32)
    pre_ref[...] = acc + b_ref[...]


def _layer_post_body(dinv_ref, g2_ref, x0_ref, pre_ref, w_ref, h_ref, xsh_ref, *, relu):
    dinv = dinv_ref[...]
    x2 = -2.0 * (g2_ref[0] + g2_ref[1]) * dinv - x0_ref[...]
    acc = pre_ref[...] + jnp.dot(x2, w_ref[2 * F:3 * F, :],
                                 preferred_element_type=jnp.float32)
    if relu:
        acc = jnp.maximum(acc, 0.0)
    h_ref[...] = acc
    if xsh_ref is not None:
        xsh_ref[...] = acc * dinv


def _tc_grid(N_pad):
    nb = 8
    B = N_pad // nb
    dinv_spec = pl.BlockSpec((B, 1), lambda i: (i, 0))
    row_spec = pl.BlockSpec((B, F), lambda i: (i, 0))
    g_spec = pl.BlockSpec((NC, B, F), lambda i: (0, i, 0))
    return nb, B, dinv_spec, row_spec, g_spec


def _prescale(deg32, x_pad):
    N_pad = x_pad.shape[0]
    nb, B, dinv_spec, row_spec, g_spec = _tc_grid(N_pad)
    deg_spec = pl.BlockSpec((NC * NS, B), lambda i: (0, i))
    return pl.pallas_call(
        _prescale_body,
        grid=(nb,),
        in_specs=[deg_spec, row_spec],
        out_specs=(row_spec, dinv_spec),
        out_shape=(
            jax.ShapeDtypeStruct((N_pad, F), jnp.float32),
            jax.ShapeDtypeStruct((N_pad, 1), jnp.float32),
        ),
    )(deg32, x_pad)


def _combine_pre(dinv, g, x0, w, b):
    N_pad = g.shape[1]
    nb, B, dinv_spec, row_spec, g_spec = _tc_grid(N_pad)
    w_spec = pl.BlockSpec((3 * F, F), lambda i: (0, 0))
    b_spec = pl.BlockSpec((1, F), lambda i: (0, 0))
    return pl.pallas_call(
        _combine_pre_body,
        grid=(nb,),
        in_specs=[dinv_spec, g_spec, row_spec, w_spec, b_spec],
        out_specs=(row_spec, row_spec),
        out_shape=(
            jax.ShapeDtypeStruct((N_pad, F), jnp.float32),
            jax.ShapeDtypeStruct((N_pad, F), jnp.float32),
        ),
    )(dinv, g, x0, w, b)


def _layer_post(dinv, g2, x0, pre, w, *, relu, need_xs):
    N_pad = x0.shape[0]
    nb, B, dinv_spec, row_spec, g_spec = _tc_grid(N_pad)
    w_spec = pl.BlockSpec((3 * F, F), lambda i: (0, 0))
    if need_xs:
        body = functools.partial(_layer_post_body, relu=relu)
        out_specs = (row_spec, row_spec)
        out_shape = (
            jax.ShapeDtypeStruct((N_pad, F), jnp.float32),
            jax.ShapeDtypeStruct((N_pad, F), jnp.float32),
        )
    else:
        def body(dinv_ref, g2_ref, x0_ref, pre_ref, w_ref, h_ref):
            _layer_post_body(dinv_ref, g2_ref, x0_ref, pre_ref, w_ref, h_ref,
                             None, relu=relu)
        out_specs = row_spec
        out_shape = jax.ShapeDtypeStruct((N_pad, F), jnp.float32)
    return pl.pallas_call(
        body,
        grid=(nb,),
        in_specs=[dinv_spec, g_spec, row_spec, row_spec, w_spec],
        out_specs=out_specs,
        out_shape=out_shape,
    )(dinv, g2, x0, pre, w)


def kernel(features, edge_index, W1, b1, W2, b2):
    N, D = features.shape
    E = edge_index.shape[1]
    N_pad, E_pad, bpt = _sizes(N, E)
    nblk = E_pad // EB

    src = edge_index[0]
    dst = edge_index[1]
    pad = N + (jnp.arange(E_pad - E, dtype=jnp.int32) % (N_pad - N))
    srcb = jnp.concatenate([src, pad]).reshape(nblk, EB)
    dstb = jnp.concatenate([dst, pad]).reshape(nblk, EB)
    x_pad = jnp.pad(features, ((0, N_pad - N), (0, 0)))

    deg_fn = _make_deg(N_pad, nblk)
    spmm_fn = _make_spmm(N_pad, nblk)

    deg32 = deg_fn(dstb)
    xs, dinv = _prescale(deg32, x_pad)
    g = spmm_fn(xs, srcb, dstb)
    xs1, pre1 = _combine_pre(dinv, g, x_pad, W1, b1.reshape(1, F))
    g2 = spmm_fn(xs1, srcb, dstb)
    h, xsh = _layer_post(dinv, g2, x_pad, pre1, W1, relu=True, need_xs=True)
    gh = spmm_fn(xsh, srcb, dstb)
    xsh1, pre2 = _combine_pre(dinv, gh, h, W2, b2.reshape(1, F))
    gh2 = spmm_fn(xsh1, srcb, dstb)
    out = _layer_post(dinv, gh2, h, pre2, W2, relu=False, need_xs=False)
    return out[:N]
